# baseline (speedup 1.0000x reference)
import numpy as np
import jax
import jax.numpy as jnp
from jax.experimental import pallas as pl
from jax.experimental.pallas import tpu as pltpu

N_KPS = 8
N_NODES = 24
ER = 64
NR = 64
EE = 32
ATTN = 32


def _edge_structure():
    src, dst = [], []
    for i in range(N_NODES):
        for j in range(N_NODES):
            if i == j:
                continue
            src.append(i)
            dst.append(j)
    src = np.array(src)
    dst = np.array(dst)
    e = len(src)
    scat = np.zeros((e, N_NODES), np.float32)
    scat[np.arange(e), src] = 1.0
    intra = (src // N_KPS == dst // N_KPS).astype(np.float32)[:, None]
    return src, dst, scat, intra


_SRC, _DST, _SCAT, _M_INTRA = _edge_structure()
N_SPATIAL = len(_SRC)


def _srnn_kernel(nodes_ref, et_ref, es_ref, scat_ref, scat_t_ref, mi_ref,
                 me_ref, seconst_ref, w_te_ref, b_te_ref, te_wih_ref,
                 te_whh_ref, te_b_ref, w_se_d_ref, w_se_l_ref, se_wih_ref,
                 se_whh_ref, se_b_ref, wq_ref, wki_ref, wke_ref, bqi_ref,
                 bqe_ref, wsi_ref, wse_ref, w_ne_ref, b_ne_ref, w_ee_t_ref,
                 w_ee_i_ref, w_ee_e_ref, b_ee_ref, nd_wih_n_ref, nd_wih_e_ref,
                 nd_whh_ref, nd_b_ref, w_out_ref, b_out_ref, out_ref):
    T = nodes_ref.shape[1]
    E = es_ref.shape[2]
    N = nodes_ref.shape[2]

    scat = scat_ref[...]
    scat_t = scat_t_ref[...]
    m_i = mi_ref[...]
    m_e = me_ref[...]
    se_const = seconst_ref[...]

    def lstm(pre, h, c, whh_ref):
        g = pre + h @ whh_ref[...]
        i = jax.nn.sigmoid(g[:, 0 * ER:1 * ER])
        f = jax.nn.sigmoid(g[:, 1 * ER:2 * ER])
        gg = jnp.tanh(g[:, 2 * ER:3 * ER])
        o = jax.nn.sigmoid(g[:, 3 * ER:4 * ER])
        c2 = f * c + i * gg
        h2 = o * jnp.tanh(c2)
        return h2, c2

    def attend(q_e, h_spat, wk_ref, bqk_ref, ws_ref, mask):
        k = h_spat @ wk_ref[...]
        s = jnp.tanh(q_e + k + bqk_ref[...]) @ ws_ref[...]
        s = s - jnp.max(s)
        ex = jnp.exp(s) * mask
        den = scat_t @ ex
        w = ex / (scat @ den)
        return scat_t @ (w * h_spat)

    def step(t, carry):
        h_temp, c_temp, h_spat, c_spat, h_node, c_node = carry

        et = et_ref[0, t]
        te_in = jax.nn.relu(et @ w_te_ref[...] + b_te_ref[...])
        h_temp, c_temp = lstm(te_in @ te_wih_ref[...] + te_b_ref[...],
                              h_temp, c_temp, te_whh_ref)

        disp = es_ref[0, t]
        dist = jnp.sqrt(jnp.sum(disp * disp, axis=1, keepdims=True))
        dist = jnp.maximum(dist, 1e-6)
        se_pre = ((disp / dist) @ w_se_d_ref[...]
                  + jnp.log(dist) * w_se_l_ref[...] + se_const)
        se_in = jax.nn.relu(se_pre)
        h_spat, c_spat = lstm(se_in @ se_wih_ref[...] + se_b_ref[...],
                              h_spat, c_spat, se_whh_ref)

        q = h_temp @ wq_ref[...]
        q_e = scat @ q
        h_ia = attend(q_e, h_spat, wki_ref, bqi_ref, wsi_ref, m_i)
        h_ea = attend(q_e, h_spat, wke_ref, bqe_ref, wse_ref, m_e)

        node_in = jax.nn.relu(nodes_ref[0, t] @ w_ne_ref[...] + b_ne_ref[...])
        edge_in = jax.nn.relu(h_temp @ w_ee_t_ref[...] + h_ia @ w_ee_i_ref[...]
                              + h_ea @ w_ee_e_ref[...] + b_ee_ref[...])
        pre_n = (node_in @ nd_wih_n_ref[...] + edge_in @ nd_wih_e_ref[...]
                 + nd_b_ref[...])
        h_node, c_node = lstm(pre_n, h_node, c_node, nd_whh_ref)

        out_ref[0, t] = h_node @ w_out_ref[...] + b_out_ref[...]
        return h_temp, c_temp, h_spat, c_spat, h_node, c_node

    z = jnp.zeros((N, ER), jnp.float32)
    ze = jnp.zeros((E, ER), jnp.float32)
    zn = jnp.zeros((N, NR), jnp.float32)
    jax.lax.fori_loop(0, T, step, (z, z, ze, ze, zn, zn))


def kernel(nodes, edges_temporal, edges_spatial, params):
    p = params
    B, T, N, _ = nodes.shape
    E = edges_spatial.shape[2]

    scat = jnp.asarray(_SCAT)
    scat_t = jnp.asarray(_SCAT.T.copy())
    m_i = jnp.asarray(_M_INTRA)
    m_e = 1.0 - m_i

    kp = p['kp_emb']
    w_se = p['W_se']
    se_const = (kp[_SRC % N_KPS] @ w_se[3:3 + N_KPS]
                + kp[_DST % N_KPS] @ w_se[3 + N_KPS:3 + 2 * N_KPS]
                + p['b_se'][None, :])

    def r2(x):
        return x.reshape(1, -1)

    weights = (
        scat, scat_t, m_i, m_e, se_const,
        p['W_te'], r2(p['b_te']),
        p['te_Wih'], p['te_Whh'], r2(p['te_bih'] + p['te_bhh']),
        w_se[0:2], w_se[2:3],
        p['se_Wih'], p['se_Whh'], r2(p['se_bih'] + p['se_bhh']),
        p['Wq'], p['Wki'], p['Wke'],
        r2(p['bq'] + p['bki']), r2(p['bq'] + p['bke']),
        p['Ws_intra'], p['Ws_inter'],
        p['W_ne'], r2(p['b_ne']),
        p['W_ee'][0:ER], p['W_ee'][ER:2 * ER], p['W_ee'][2 * ER:3 * ER],
        r2(p['b_ee']),
        p['nd_Wih'][0:EE], p['nd_Wih'][EE:2 * EE],
        p['nd_Whh'], r2(p['nd_bih'] + p['nd_bhh']),
        p['W_out'], r2(p['b_out']),
    )

    def full(x):
        return pl.BlockSpec(x.shape, lambda b: (0,) * x.ndim)

    in_specs = [
        pl.BlockSpec((1, T, N, 2), lambda b: (b, 0, 0, 0)),
        pl.BlockSpec((1, T, N, 2), lambda b: (b, 0, 0, 0)),
        pl.BlockSpec((1, T, E, 2), lambda b: (b, 0, 0, 0)),
    ] + [full(w) for w in weights]

    out = pl.pallas_call(
        _srnn_kernel,
        grid=(B,),
        in_specs=in_specs,
        out_specs=pl.BlockSpec((1, T, N, 5), lambda b: (b, 0, 0, 0)),
        out_shape=jax.ShapeDtypeStruct((B, T, N, 5), jnp.float32),
        compiler_params=pltpu.CompilerParams(
            dimension_semantics=("parallel",)),
    )(nodes, edges_temporal, edges_spatial, *weights)
    return out


# speedup vs baseline: 3.1486x; 3.1486x over previous
import numpy as np
import jax
import jax.numpy as jnp
from jax.experimental import pallas as pl
from jax.experimental.pallas import tpu as pltpu

N_KPS = 8
N_NODES = 24
ER = 64
NR = 64
EE = 32
ATTN = 32
NEG = -1e30


def _edge_structure():
    gather_idx = np.zeros((N_NODES * N_NODES,), np.int32)
    src, dst = [], []
    e = 0
    for i in range(N_NODES):
        for j in range(N_NODES):
            if i == j:
                continue
            gather_idx[i * N_NODES + j] = e
            src.append(i)
            dst.append(j)
            e += 1
    madd = np.full((N_NODES, N_NODES, 2), NEG, np.float32)
    for n in range(N_NODES):
        for j in range(N_NODES):
            if j == n:
                continue
            if j // N_KPS == n // N_KPS:
                madd[n, j, 0] = 0.0
            else:
                madd[n, j, 1] = 0.0
    return np.array(src), np.array(dst), gather_idx, madd


_SRC, _DST, _GATHER, _MADD = _edge_structure()


def _srnn_kernel(nodes_ref, et_ref, es_ref, madd_ref, seconst_ref, w_te_ref,
                 b_te_ref, te_wih_ref, te_whh_ref, te_b_ref, w_se_d_ref,
                 w_se_l_ref, se_wih_ref, se_whh_ref, se_b_ref, wq2_ref,
                 wk2_ref, bqk2_ref, ws2_ref, w_ne_ref, b_ne_ref, w_ee_t_ref,
                 w_ee_i_ref, w_ee_e_ref, b_ee_ref, nd_wih_n_ref, nd_wih_e_ref,
                 nd_whh_ref, nd_b_ref, w_out_ref, b_out_ref, out_ref,
                 ht_ref, ct_ref, hs_ref, cs_ref, hn_ref, cn_ref):
    G = nodes_ref.shape[1]
    E = es_ref.shape[1]

    @pl.when(pl.program_id(0) == 0)
    def _init():
        ht_ref[...] = jnp.zeros_like(ht_ref)
        ct_ref[...] = jnp.zeros_like(ct_ref)
        hs_ref[...] = jnp.zeros_like(hs_ref)
        cs_ref[...] = jnp.zeros_like(cs_ref)
        hn_ref[...] = jnp.zeros_like(hn_ref)
        cn_ref[...] = jnp.zeros_like(cn_ref)

    def lstm(pre, h, c, whh_ref):
        g = pre + h @ whh_ref[...]
        i = jax.nn.sigmoid(g[:, 0 * ER:1 * ER])
        f = jax.nn.sigmoid(g[:, 1 * ER:2 * ER])
        gg = jnp.tanh(g[:, 2 * ER:3 * ER])
        o = jax.nn.sigmoid(g[:, 3 * ER:4 * ER])
        c2 = f * c + i * gg
        h2 = o * jnp.tanh(c2)
        return h2, c2

    et = et_ref[0]
    te_in = jax.nn.relu(et @ w_te_ref[...] + b_te_ref[...])
    h_temp, c_temp = lstm(te_in @ te_wih_ref[...] + te_b_ref[...],
                          ht_ref[...], ct_ref[...], te_whh_ref)
    ht_ref[...] = h_temp
    ct_ref[...] = c_temp

    disp = es_ref[0]
    dist = jnp.sqrt(jnp.sum(disp * disp, axis=1, keepdims=True))
    dist = jnp.maximum(dist, 1e-6)
    se_pre = ((disp / dist) @ w_se_d_ref[...]
              + jnp.log(dist) * w_se_l_ref[...] + seconst_ref[...])
    se_in = jax.nn.relu(se_pre)
    h_spat, c_spat = lstm(se_in @ se_wih_ref[...] + se_b_ref[...],
                          hs_ref[...], cs_ref[...], se_whh_ref)
    hs_ref[...] = h_spat
    cs_ref[...] = c_spat

    q2 = h_temp @ wq2_ref[...]
    k2 = h_spat @ wk2_ref[...]
    k2 = k2.reshape(G, N_NODES, 2 * ATTN)
    u2 = jnp.tanh(q2[:, None, :] + k2 + bqk2_ref[...])
    s2 = u2.reshape(E, 2 * ATTN) @ ws2_ref[...]
    s2 = s2.reshape(G, N_NODES, 2) + madd_ref[...]
    s2 = s2 - jnp.max(s2, axis=1, keepdims=True)
    ex = jnp.exp(s2)
    w2 = ex / jnp.sum(ex, axis=1, keepdims=True)
    hs3 = h_spat.reshape(G, N_NODES, ER)
    h_ia = jnp.sum(w2[:, :, 0:1] * hs3, axis=1)
    h_ea = jnp.sum(w2[:, :, 1:2] * hs3, axis=1)

    node_in = jax.nn.relu(nodes_ref[0] @ w_ne_ref[...] + b_ne_ref[...])
    edge_in = jax.nn.relu(h_temp @ w_ee_t_ref[...] + h_ia @ w_ee_i_ref[...]
                          + h_ea @ w_ee_e_ref[...] + b_ee_ref[...])
    pre_n = (node_in @ nd_wih_n_ref[...] + edge_in @ nd_wih_e_ref[...]
             + nd_b_ref[...])
    h_node, c_node = lstm(pre_n, hn_ref[...], cn_ref[...], nd_whh_ref)
    hn_ref[...] = h_node
    cn_ref[...] = c_node

    out_ref[0] = h_node @ w_out_ref[...] + b_out_ref[...]


def kernel(nodes, edges_temporal, edges_spatial, params):
    p = params
    B, T, N, _ = nodes.shape
    G = B * N
    E = B * N * N

    nodes_t = nodes.transpose(1, 0, 2, 3).reshape(T, G, 2)
    et_t = edges_temporal.transpose(1, 0, 2, 3).reshape(T, G, 2)
    es_p = jnp.take(edges_spatial, jnp.asarray(_GATHER), axis=2)
    es_t = es_p.transpose(1, 0, 2, 3).reshape(T, E, 2)
    madd = jnp.asarray(np.tile(_MADD, (B, 1, 1)))

    kp = p['kp_emb']
    w_se = p['W_se']
    se_const0 = (kp[_SRC % N_KPS] @ w_se[3:3 + N_KPS]
                 + kp[_DST % N_KPS] @ w_se[3 + N_KPS:3 + 2 * N_KPS]
                 + p['b_se'][None, :])
    se_const = jnp.tile(se_const0[jnp.asarray(_GATHER)], (B, 1))

    def r2(x):
        return x.reshape(1, -1)

    wq2 = jnp.concatenate([p['Wq'], p['Wq']], axis=1)
    wk2 = jnp.concatenate([p['Wki'], p['Wke']], axis=1)
    bqk2 = (jnp.concatenate([p['bq'] + p['bki'], p['bq'] + p['bke']])
            .reshape(1, 1, 2 * ATTN))
    ws2 = jnp.zeros((2 * ATTN, 2), jnp.float32)
    ws2 = ws2.at[:ATTN, 0:1].set(p['Ws_intra']).at[ATTN:, 1:2].set(p['Ws_inter'])

    weights = (
        madd, se_const,
        p['W_te'], r2(p['b_te']),
        p['te_Wih'], p['te_Whh'], r2(p['te_bih'] + p['te_bhh']),
        w_se[0:2], w_se[2:3],
        p['se_Wih'], p['se_Whh'], r2(p['se_bih'] + p['se_bhh']),
        wq2, wk2, bqk2, ws2,
        p['W_ne'], r2(p['b_ne']),
        p['W_ee'][0:ER], p['W_ee'][ER:2 * ER], p['W_ee'][2 * ER:3 * ER],
        r2(p['b_ee']),
        p['nd_Wih'][0:EE], p['nd_Wih'][EE:2 * EE],
        p['nd_Whh'], r2(p['nd_bih'] + p['nd_bhh']),
        p['W_out'], r2(p['b_out']),
    )

    def full(x):
        nd = x.ndim
        return pl.BlockSpec(x.shape, lambda t, _n=nd: (0,) * _n)

    in_specs = [
        pl.BlockSpec((1, G, 2), lambda t: (t, 0, 0)),
        pl.BlockSpec((1, G, 2), lambda t: (t, 0, 0)),
        pl.BlockSpec((1, E, 2), lambda t: (t, 0, 0)),
    ] + [full(w) for w in weights]

    out = pl.pallas_call(
        _srnn_kernel,
        grid=(T,),
        in_specs=in_specs,
        out_specs=pl.BlockSpec((1, G, 5), lambda t: (t, 0, 0)),
        out_shape=jax.ShapeDtypeStruct((T, G, 5), jnp.float32),
        scratch_shapes=[
            pltpu.VMEM((G, ER), jnp.float32),
            pltpu.VMEM((G, ER), jnp.float32),
            pltpu.VMEM((E, ER), jnp.float32),
            pltpu.VMEM((E, ER), jnp.float32),
            pltpu.VMEM((G, NR), jnp.float32),
            pltpu.VMEM((G, NR), jnp.float32),
        ],
        compiler_params=pltpu.CompilerParams(
            dimension_semantics=("arbitrary",)),
    )(nodes_t, et_t, es_t, *weights)
    return out.reshape(T, B, N, 5).transpose(1, 0, 2, 3)


# speedup vs baseline: 4.3451x; 1.3800x over previous
import numpy as np
import jax
import jax.numpy as jnp
from jax.experimental import pallas as pl
from jax.experimental.pallas import tpu as pltpu

N_KPS = 8
N_NODES = 24
ER = 64
NR = 64
EE = 32
ATTN = 32
NEG = -1e30


def _edge_structure():
    gather_idx = np.zeros((N_NODES * N_NODES,), np.int32)
    src, dst = [], []
    e = 0
    for i in range(N_NODES):
        for j in range(N_NODES):
            if i == j:
                continue
            gather_idx[i * N_NODES + j] = e
            src.append(i)
            dst.append(j)
            e += 1
    madd = np.full((N_NODES, N_NODES, 2), NEG, np.float32)
    for n in range(N_NODES):
        for j in range(N_NODES):
            if j == n:
                continue
            if j // N_KPS == n // N_KPS:
                madd[n, j, 0] = 0.0
            else:
                madd[n, j, 1] = 0.0
    return np.array(src), np.array(dst), gather_idx, madd


_SRC, _DST, _GATHER, _MADD = _edge_structure()


def _srnn_kernel(nodes_ref, et_ref, es_ref, madd_ref, seconst_ref, w_te_ref,
                 b_te_ref, te_wih_ref, te_whh_ref, te_b_ref, w3_ref,
                 se_wih_ref, se_whh_ref, se_b_ref, wq2_ref,
                 wk2_ref, bqk2_ref, ws2_ref, w_ne_ref, b_ne_ref, w_ee_t_ref,
                 w_ee_i_ref, w_ee_e_ref, b_ee_ref, nd_wih_n_ref, nd_wih_e_ref,
                 nd_whh_ref, nd_b_ref, w_out_ref, b_out_ref, out_ref,
                 ht_ref, ct_ref, hs_ref, cs_ref, hn_ref, cn_ref):
    B = nodes_ref.shape[0]
    N = nodes_ref.shape[2]
    G = B * N
    E = es_ref.shape[2]

    @pl.when(pl.program_id(0) == 0)
    def _init():
        ht_ref[...] = jnp.zeros_like(ht_ref)
        ct_ref[...] = jnp.zeros_like(ct_ref)
        hs_ref[...] = jnp.zeros_like(hs_ref)
        cs_ref[...] = jnp.zeros_like(cs_ref)
        hn_ref[...] = jnp.zeros_like(hn_ref)
        cn_ref[...] = jnp.zeros_like(cn_ref)

    def lstm(pre, h, c, whh_ref):
        g = pre + h @ whh_ref[...]
        i = jax.nn.sigmoid(g[:, 0 * ER:1 * ER])
        f = jax.nn.sigmoid(g[:, 1 * ER:2 * ER])
        gg = jnp.tanh(g[:, 2 * ER:3 * ER])
        o = jax.nn.sigmoid(g[:, 3 * ER:4 * ER])
        c2 = f * c + i * gg
        h2 = o * jnp.tanh(c2)
        return h2, c2

    et = et_ref[...].reshape(G, 2)
    te_in = jax.nn.relu(et @ w_te_ref[...] + b_te_ref[...])
    h_temp, c_temp = lstm(te_in @ te_wih_ref[...] + te_b_ref[...],
                          ht_ref[...], ct_ref[...], te_whh_ref)
    ht_ref[...] = h_temp
    ct_ref[...] = c_temp

    esr = es_ref[0]
    d2 = jnp.maximum(esr[0:1] * esr[0:1] + esr[1:2] * esr[1:2], 1e-12)
    feat_t = jnp.concatenate(
        [esr * jax.lax.rsqrt(d2), 0.5 * jnp.log(d2)], axis=0)
    se_pre = jax.lax.dot_general(
        feat_t, w3_ref[...], (((0,), (0,)), ((), ()))) + seconst_ref[...]
    se_in = jax.nn.relu(se_pre)
    h_spat, c_spat = lstm(se_in @ se_wih_ref[...] + se_b_ref[...],
                          hs_ref[...], cs_ref[...], se_whh_ref)
    hs_ref[...] = h_spat
    cs_ref[...] = c_spat

    q2 = h_temp @ wq2_ref[...]
    k2 = h_spat @ wk2_ref[...]
    k2 = k2.reshape(G, N_NODES, 2 * ATTN)
    u2 = jnp.tanh(q2[:, None, :] + k2 + bqk2_ref[...])
    s2 = u2.reshape(E, 2 * ATTN) @ ws2_ref[...]
    s3 = s2.reshape(G, N_NODES, 2) + madd_ref[...]
    ex = jnp.exp(s3)
    w2 = ex / jnp.sum(ex, axis=1, keepdims=True)
    hs3 = h_spat.reshape(G, N_NODES, ER)
    h_ia = jnp.sum(w2[:, :, 0:1] * hs3, axis=1)
    h_ea = jnp.sum(w2[:, :, 1:2] * hs3, axis=1)

    node_in = jax.nn.relu(nodes_ref[...].reshape(G, 2) @ w_ne_ref[...]
                          + b_ne_ref[...])
    edge_in = jax.nn.relu(h_temp @ w_ee_t_ref[...] + h_ia @ w_ee_i_ref[...]
                          + h_ea @ w_ee_e_ref[...] + b_ee_ref[...])
    pre_n = (node_in @ nd_wih_n_ref[...] + edge_in @ nd_wih_e_ref[...]
             + nd_b_ref[...])
    h_node, c_node = lstm(pre_n, hn_ref[...], cn_ref[...], nd_whh_ref)
    hn_ref[...] = h_node
    cn_ref[...] = c_node

    res = h_node @ w_out_ref[...] + b_out_ref[...]
    out_ref[...] = res.reshape(B, 1, N, 5)


def kernel(nodes, edges_temporal, edges_spatial, params):
    p = params
    B, T, N, _ = nodes.shape
    G = B * N
    E = B * N * N

    es_p = jnp.take(edges_spatial, jnp.asarray(_GATHER), axis=2)
    es_t = es_p.transpose(1, 3, 0, 2).reshape(T, 2, E)
    base = np.tile(_MADD, (B, 1, 1))
    shift = jnp.stack([jnp.sum(jnp.abs(p['Ws_intra'])),
                       jnp.sum(jnp.abs(p['Ws_inter']))])
    madd = jnp.asarray(base) - (base == 0.0) * shift[None, None, :]

    kp = p['kp_emb']
    w_se = p['W_se']
    se_const0 = (kp[_SRC % N_KPS] @ w_se[3:3 + N_KPS]
                 + kp[_DST % N_KPS] @ w_se[3 + N_KPS:3 + 2 * N_KPS]
                 + p['b_se'][None, :])
    se_const = jnp.tile(se_const0[jnp.asarray(_GATHER)], (B, 1))

    def r2(x):
        return x.reshape(1, -1)

    wq2 = jnp.concatenate([p['Wq'], p['Wq']], axis=1)
    wk2 = jnp.concatenate([p['Wki'], p['Wke']], axis=1)
    bqk2 = (jnp.concatenate([p['bq'] + p['bki'], p['bq'] + p['bke']])
            .reshape(1, 1, 2 * ATTN))
    ws2 = jnp.zeros((2 * ATTN, 2), jnp.float32)
    ws2 = ws2.at[:ATTN, 0:1].set(p['Ws_intra']).at[ATTN:, 1:2].set(p['Ws_inter'])

    weights = (
        madd, se_const,
        p['W_te'], r2(p['b_te']),
        p['te_Wih'], p['te_Whh'], r2(p['te_bih'] + p['te_bhh']),
        w_se[0:3],
        p['se_Wih'], p['se_Whh'], r2(p['se_bih'] + p['se_bhh']),
        wq2, wk2, bqk2, ws2,
        p['W_ne'], r2(p['b_ne']),
        p['W_ee'][0:ER], p['W_ee'][ER:2 * ER], p['W_ee'][2 * ER:3 * ER],
        r2(p['b_ee']),
        p['nd_Wih'][0:EE], p['nd_Wih'][EE:2 * EE],
        p['nd_Whh'], r2(p['nd_bih'] + p['nd_bhh']),
        p['W_out'], r2(p['b_out']),
    )

    def full(x):
        nd = x.ndim
        return pl.BlockSpec(x.shape, lambda t, _n=nd: (0,) * _n)

    in_specs = [
        pl.BlockSpec((B, 1, N, 2), lambda t: (0, t, 0, 0)),
        pl.BlockSpec((B, 1, N, 2), lambda t: (0, t, 0, 0)),
        pl.BlockSpec((1, 2, E), lambda t: (t, 0, 0)),
    ] + [full(w) for w in weights]

    out = pl.pallas_call(
        _srnn_kernel,
        grid=(T,),
        in_specs=in_specs,
        out_specs=pl.BlockSpec((B, 1, N, 5), lambda t: (0, t, 0, 0)),
        out_shape=jax.ShapeDtypeStruct((B, T, N, 5), jnp.float32),
        scratch_shapes=[
            pltpu.VMEM((G, ER), jnp.float32),
            pltpu.VMEM((G, ER), jnp.float32),
            pltpu.VMEM((E, ER), jnp.float32),
            pltpu.VMEM((E, ER), jnp.float32),
            pltpu.VMEM((G, NR), jnp.float32),
            pltpu.VMEM((G, NR), jnp.float32),
        ],
        compiler_params=pltpu.CompilerParams(
            dimension_semantics=("arbitrary",)),
    )(nodes, edges_temporal, es_t, *weights)
    return out
